# baseline (speedup 1.0000x reference)
import functools

import jax
import jax.numpy as jnp
from jax.experimental import pallas as pl
from jax.experimental.pallas import tpu as pltpu

_EPS = 1e-5
_NBUF = 4
_PREFETCH = 2
_NSPLIT = 2


def _compute(x, t, gam, bet, num_types):
    mean = jnp.mean(x, axis=1, keepdims=True)
    xc = x - mean
    var = jnp.mean(xc * xc, axis=1, keepdims=True)
    xhat = xc * jax.lax.rsqrt(var + _EPS)
    onehot = (t == jax.lax.broadcasted_iota(
        jnp.int32, (t.shape[0], num_types), 1)).astype(jnp.float32)
    g = jnp.dot(onehot, gam, preferred_element_type=jnp.float32)
    b = jnp.dot(onehot, bet, preferred_element_type=jnp.float32)
    return xhat * g + b


def _ring_body(t_hbm, x_hbm, g_ref, b_ref, o_hbm,
               x_buf, t_buf, o_buf, in_sem, t_sem, out_sem,
               *, num_types, block, steps):
    gam = g_ref[...]
    bet = b_ref[...]
    base = pl.program_id(0) * (steps * block)

    def start_in(slot, step):
        pltpu.make_async_copy(
            x_hbm.at[pl.ds(base + step * block, block)], x_buf.at[slot],
            in_sem.at[slot]).start()
        pltpu.make_async_copy(
            t_hbm.at[pl.ds(base + step * block, block)], t_buf.at[slot],
            t_sem.at[slot]).start()

    def wait_in(slot):
        pltpu.make_async_copy(
            x_hbm.at[pl.ds(0, block)], x_buf.at[slot],
            in_sem.at[slot]).wait()
        pltpu.make_async_copy(
            t_hbm.at[pl.ds(0, block)], t_buf.at[slot],
            t_sem.at[slot]).wait()

    def start_out(slot, step):
        pltpu.make_async_copy(
            o_buf.at[slot], o_hbm.at[pl.ds(base + step * block, block)],
            out_sem.at[slot]).start()

    def wait_out(slot):
        pltpu.make_async_copy(
            o_buf.at[slot], o_hbm.at[pl.ds(0, block)],
            out_sem.at[slot]).wait()

    for s in range(_PREFETCH):
        start_in(s % _NBUF, s)

    def body(step, carry):
        slot = jax.lax.rem(step, _NBUF)

        @pl.when(step >= _NBUF)
        def _():
            wait_out(slot)

        @pl.when(step + _PREFETCH < steps)
        def _():
            start_in(jax.lax.rem(step + _PREFETCH, _NBUF), step + _PREFETCH)

        wait_in(slot)
        x = x_buf[slot]
        t = t_buf[slot]
        o_buf[slot] = _compute(x, t, gam, bet, num_types)
        start_out(slot, step)
        return carry

    jax.lax.fori_loop(0, steps, body, 0)

    for s in range(max(steps - _NBUF, 0), steps):
        wait_out(s % _NBUF)


def kernel(type_list, abstract_features, gamma, beta):
    n, d = abstract_features.shape
    num_types = gamma.shape[0]
    t2 = type_list.astype(jnp.int32).reshape(n, 1)

    nn = n // _NSPLIT
    block = nn
    for cand in (8000, 5000, 4000, 2500, 2000, 1000, 500, 250, 200, 100):
        if nn % cand == 0:
            block = cand
            break
    steps = nn // block

    return pl.pallas_call(
        functools.partial(_ring_body, num_types=num_types, block=block,
                          steps=steps),
        out_shape=jax.ShapeDtypeStruct((n, d), jnp.float32),
        grid=(_NSPLIT,),
        in_specs=[
            pl.BlockSpec(memory_space=pltpu.MemorySpace.HBM),
            pl.BlockSpec(memory_space=pltpu.MemorySpace.HBM),
            pl.BlockSpec(memory_space=pltpu.VMEM),
            pl.BlockSpec(memory_space=pltpu.VMEM),
        ],
        out_specs=pl.BlockSpec(memory_space=pltpu.MemorySpace.HBM),
        scratch_shapes=[
            pltpu.VMEM((_NBUF, block, d), jnp.float32),
            pltpu.VMEM((_NBUF, block, 1), jnp.int32),
            pltpu.VMEM((_NBUF, block, d), jnp.float32),
            pltpu.SemaphoreType.DMA((_NBUF,)),
            pltpu.SemaphoreType.DMA((_NBUF,)),
            pltpu.SemaphoreType.DMA((_NBUF,)),
        ],
        compiler_params=pltpu.CompilerParams(
            dimension_semantics=("parallel",),
        ),
        name="typenorm_ring2",
    )(t2, abstract_features, gamma, beta)


# speedup vs baseline: 1.0013x; 1.0013x over previous
import functools

import jax
import jax.numpy as jnp
from jax.experimental import pallas as pl
from jax.experimental.pallas import tpu as pltpu

_EPS = 1e-5
_NBUF = 4
_PREFETCH = 2
_NSPLIT = 1


def _compute(x, t, gam, bet, num_types):
    mean = jnp.mean(x, axis=1, keepdims=True)
    xc = x - mean
    var = jnp.mean(xc * xc, axis=1, keepdims=True)
    xhat = xc * jax.lax.rsqrt(var + _EPS)
    onehot = (t == jax.lax.broadcasted_iota(
        jnp.int32, (t.shape[0], num_types), 1)).astype(jnp.float32)
    g = jnp.dot(onehot, gam, preferred_element_type=jnp.float32)
    b = jnp.dot(onehot, bet, preferred_element_type=jnp.float32)
    return xhat * g + b


def _ring_body(t_hbm, x_hbm, g_ref, b_ref, o_hbm,
               x_buf, t_buf, o_buf, in_sem, t_sem, out_sem,
               *, num_types, block, steps):
    gam = g_ref[...]
    bet = b_ref[...]
    base = pl.program_id(0) * (steps * block)

    def start_in(slot, step):
        pltpu.make_async_copy(
            x_hbm.at[pl.ds(base + step * block, block)], x_buf.at[slot],
            in_sem.at[slot]).start()
        pltpu.make_async_copy(
            t_hbm.at[pl.ds(base + step * block, block)], t_buf.at[slot],
            t_sem.at[slot]).start()

    def wait_in(slot):
        pltpu.make_async_copy(
            x_hbm.at[pl.ds(0, block)], x_buf.at[slot],
            in_sem.at[slot]).wait()
        pltpu.make_async_copy(
            t_hbm.at[pl.ds(0, block)], t_buf.at[slot],
            t_sem.at[slot]).wait()

    def start_out(slot, step):
        pltpu.make_async_copy(
            o_buf.at[slot], o_hbm.at[pl.ds(base + step * block, block)],
            out_sem.at[slot]).start()

    def wait_out(slot):
        pltpu.make_async_copy(
            o_buf.at[slot], o_hbm.at[pl.ds(0, block)],
            out_sem.at[slot]).wait()

    for s in range(_PREFETCH):
        start_in(s % _NBUF, s)

    def body(step, carry):
        slot = jax.lax.rem(step, _NBUF)

        @pl.when(step >= _NBUF)
        def _():
            wait_out(slot)

        @pl.when(step + _PREFETCH < steps)
        def _():
            start_in(jax.lax.rem(step + _PREFETCH, _NBUF), step + _PREFETCH)

        wait_in(slot)
        x = x_buf[slot]
        t = t_buf[slot]
        o_buf[slot] = _compute(x, t, gam, bet, num_types)
        start_out(slot, step)
        return carry

    jax.lax.fori_loop(0, steps, body, 0)

    for s in range(max(steps - _NBUF, 0), steps):
        wait_out(s % _NBUF)


def kernel(type_list, abstract_features, gamma, beta):
    n, d = abstract_features.shape
    num_types = gamma.shape[0]
    t2 = type_list.astype(jnp.int32).reshape(n, 1)

    nn = n // _NSPLIT
    block = nn
    for cand in (8000, 5000, 4000, 2500, 2000, 1000, 500, 250, 200, 100):
        if nn % cand == 0:
            block = cand
            break
    steps = nn // block

    return pl.pallas_call(
        functools.partial(_ring_body, num_types=num_types, block=block,
                          steps=steps),
        out_shape=jax.ShapeDtypeStruct((n, d), jnp.float32),
        grid=(_NSPLIT,),
        in_specs=[
            pl.BlockSpec(memory_space=pltpu.MemorySpace.HBM),
            pl.BlockSpec(memory_space=pltpu.MemorySpace.HBM),
            pl.BlockSpec(memory_space=pltpu.VMEM),
            pl.BlockSpec(memory_space=pltpu.VMEM),
        ],
        out_specs=pl.BlockSpec(memory_space=pltpu.MemorySpace.HBM),
        scratch_shapes=[
            pltpu.VMEM((_NBUF, block, d), jnp.float32),
            pltpu.VMEM((_NBUF, block, 1), jnp.int32),
            pltpu.VMEM((_NBUF, block, d), jnp.float32),
            pltpu.SemaphoreType.DMA((_NBUF,)),
            pltpu.SemaphoreType.DMA((_NBUF,)),
            pltpu.SemaphoreType.DMA((_NBUF,)),
        ],
        compiler_params=pltpu.CompilerParams(
            dimension_semantics=("parallel",),
        ),
        name="typenorm_ring2",
    )(t2, abstract_features, gamma, beta)


# speedup vs baseline: 1.0020x; 1.0007x over previous
import functools

import jax
import jax.numpy as jnp
from jax.experimental import pallas as pl
from jax.experimental.pallas import tpu as pltpu

_EPS = 1e-5
_NBUF = 4
_PREFETCH = 2
_NSPLIT = 1


def _compute(x, t, gam, bet, num_types):
    mean = jnp.mean(x, axis=1, keepdims=True)
    xc = x - mean
    var = jnp.mean(xc * xc, axis=1, keepdims=True)
    xhat = xc * jax.lax.rsqrt(var + _EPS)
    onehot = (t == jax.lax.broadcasted_iota(
        jnp.int32, (t.shape[0], num_types), 1)).astype(jnp.float32)
    g = jnp.dot(onehot, gam, preferred_element_type=jnp.float32)
    b = jnp.dot(onehot, bet, preferred_element_type=jnp.float32)
    return xhat * g + b


def _ring_body(t_hbm, x_hbm, g_ref, b_ref, o_hbm,
               x_buf, t_buf, o_buf, in_sem, t_sem, out_sem,
               *, num_types, block, steps):
    gam = g_ref[...]
    bet = b_ref[...]
    base = pl.program_id(0) * (steps * block)

    def start_in(slot, step):
        pltpu.make_async_copy(
            x_hbm.at[pl.ds(base + step * block, block)], x_buf.at[slot],
            in_sem.at[slot]).start()
        pltpu.make_async_copy(
            t_hbm.at[pl.ds(base + step * block, block)], t_buf.at[slot],
            t_sem.at[slot]).start()

    def wait_in(slot):
        pltpu.make_async_copy(
            x_hbm.at[pl.ds(0, block)], x_buf.at[slot],
            in_sem.at[slot]).wait()
        pltpu.make_async_copy(
            t_hbm.at[pl.ds(0, block)], t_buf.at[slot],
            t_sem.at[slot]).wait()

    def start_out(slot, step):
        pltpu.make_async_copy(
            o_buf.at[slot], o_hbm.at[pl.ds(base + step * block, block)],
            out_sem.at[slot]).start()

    def wait_out(slot):
        pltpu.make_async_copy(
            o_buf.at[slot], o_hbm.at[pl.ds(0, block)],
            out_sem.at[slot]).wait()

    for s in range(min(_PREFETCH, steps)):
        start_in(s % _NBUF, s)

    def body(step, carry):
        slot = jax.lax.rem(step, _NBUF)

        @pl.when(step >= _NBUF)
        def _():
            wait_out(slot)

        @pl.when(step + _PREFETCH < steps)
        def _():
            start_in(jax.lax.rem(step + _PREFETCH, _NBUF), step + _PREFETCH)

        wait_in(slot)
        x = x_buf[slot]
        t = t_buf[slot]
        o_buf[slot] = _compute(x, t, gam, bet, num_types)
        start_out(slot, step)
        return carry

    jax.lax.fori_loop(0, steps, body, 0)

    for s in range(max(steps - _NBUF, 0), steps):
        wait_out(s % _NBUF)


def kernel(type_list, abstract_features, gamma, beta):
    n, d = abstract_features.shape
    num_types = gamma.shape[0]
    t2 = type_list.astype(jnp.int32).reshape(n, 1)

    nn = n // _NSPLIT
    block = nn
    for cand in (8000, 5000, 4000, 2500, 2000, 1000, 500, 250, 200, 100):
        if nn % cand == 0:
            block = cand
            break
    steps = nn // block

    return pl.pallas_call(
        functools.partial(_ring_body, num_types=num_types, block=block,
                          steps=steps),
        out_shape=jax.ShapeDtypeStruct((n, d), jnp.float32),
        grid=(_NSPLIT,),
        in_specs=[
            pl.BlockSpec(memory_space=pltpu.MemorySpace.HBM),
            pl.BlockSpec(memory_space=pltpu.MemorySpace.HBM),
            pl.BlockSpec(memory_space=pltpu.VMEM),
            pl.BlockSpec(memory_space=pltpu.VMEM),
        ],
        out_specs=pl.BlockSpec(memory_space=pltpu.MemorySpace.HBM),
        scratch_shapes=[
            pltpu.VMEM((_NBUF, block, d), jnp.float32),
            pltpu.VMEM((_NBUF, block, 1), jnp.int32),
            pltpu.VMEM((_NBUF, block, d), jnp.float32),
            pltpu.SemaphoreType.DMA((_NBUF,)),
            pltpu.SemaphoreType.DMA((_NBUF,)),
            pltpu.SemaphoreType.DMA((_NBUF,)),
        ],
        compiler_params=pltpu.CompilerParams(
            dimension_semantics=("parallel",),
        ),
        name="typenorm_ring2",
    )(t2, abstract_features, gamma, beta)
